# baseline (speedup 1.0000x reference)
import functools

import jax
import jax.numpy as jnp
from jax import lax
from jax.experimental import pallas as pl
from jax.experimental.pallas import tpu as pltpu
from jax.experimental.pallas import tpu_sc as plsc

N_NODES = 10000
N_EDGES = 160000
D = 256
EPS = 1e-5

NC, NS, L = 2, 16, 16
NW = NC * NS


TILE_N = 400


def _k0a_body(h_ref, wa_ref, wb_ref, oa_ref, ob_ref):
    h = h_ref[...]
    oa_ref[...] = jnp.dot(h, wa_ref[...], preferred_element_type=jnp.float32)
    ob_ref[...] = jnp.dot(h, wb_ref[...], preferred_element_type=jnp.float32)


def _k0a(h, wa, wb):
    return pl.pallas_call(
        _k0a_body,
        grid=(N_NODES // TILE_N,),
        in_specs=[
            pl.BlockSpec((TILE_N, D), lambda i: (i, 0)),
            pl.BlockSpec((D, D), lambda i: (0, 0)),
            pl.BlockSpec((D, D), lambda i: (0, 0)),
        ],
        out_specs=[
            pl.BlockSpec((TILE_N, D), lambda i: (i, 0)),
            pl.BlockSpec((TILE_N, D), lambda i: (i, 0)),
        ],
        out_shape=[
            jax.ShapeDtypeStruct((N_NODES, D), jnp.float32),
            jax.ShapeDtypeStruct((N_NODES, D), jnp.float32),
        ],
    )(h, wa, wb)


TILE_E = 1600


def _k0b_body(ea_ref, we_ref, pk_ref, o_ref):
    o_ref[...] = (
        jnp.dot(ea_ref[...], we_ref[...], preferred_element_type=jnp.float32)
        + pk_ref[0:1, :]
    )


def _k0b(ea, we, pk):
    return pl.pallas_call(
        _k0b_body,
        grid=(N_EDGES // TILE_E,),
        in_specs=[
            pl.BlockSpec((TILE_E, 16), lambda i: (i, 0)),
            pl.BlockSpec((16, D), lambda i: (0, 0)),
            pl.BlockSpec((8, D), lambda i: (0, 0)),
        ],
        out_specs=pl.BlockSpec((TILE_E, D), lambda i: (i, 0)),
        out_shape=jax.ShapeDtypeStruct((N_EDGES, D), jnp.float32),
    )(ea, we, pk)


CH1 = 32
TOT_CH1 = N_EDGES // CH1
BASE_CH1 = TOT_CH1 // NW
REM_CH1 = TOT_CH1 % NW


def _k1_body(hwa, hwb, eterm, dsti, srci, w1r, posx, posy, posz,
             x1_o, pdx_o, pdy_o, pdz_o, s1_o, q1_o,
             dbuf, sbuf, bufa, bufb, bufe, xbuf, pdxb, pdyb, pdzb,
             pxv, pyv, pzv, wrv, ssum, qsum,
             sema, semb, seme):
    cid = lax.axis_index("c")
    sid = lax.axis_index("s")
    wid = sid * NC + cid

    pltpu.sync_copy(w1r, wrv)
    pltpu.sync_copy(posx, pxv)
    pltpu.sync_copy(posy, pyv)
    pltpu.sync_copy(posz, pzv)
    wrv_vals = [wrv[v] for v in range(L)]
    zero = jnp.zeros((L,), jnp.float32)
    init = (tuple(zero for _ in range(L)), tuple(zero for _ in range(L)))

    def chunk(ci, carry):
        ss, qq = carry
        r0 = (wid + ci * NW) * CH1
        pltpu.sync_copy(dsti.at[pl.ds(r0, CH1)], dbuf)
        pltpu.sync_copy(srci.at[pl.ds(r0, CH1)], sbuf)
        cpa = pltpu.async_copy(hwa.at[dbuf], bufa, sema)
        cpb = pltpu.async_copy(hwb.at[sbuf], bufb, semb)
        cpe = pltpu.async_copy(eterm.at[pl.ds(r0, CH1)], bufe, seme)
        cpa.wait()
        cpb.wait()
        cpe.wait()

        ss = list(ss)
        qq = list(qq)
        for g in range(CH1 // L):
            gsl = pl.ds(g * L, L)
            dvi = dbuf[gsl]
            svi = sbuf[gsl]
            dvx = plsc.load_gather(pxv, [dvi]) - plsc.load_gather(pxv, [svi])
            dvy = plsc.load_gather(pyv, [dvi]) - plsc.load_gather(pyv, [svi])
            dvz = plsc.load_gather(pzv, [dvi]) - plsc.load_gather(pzv, [svi])
            pdxb[gsl] = dvx
            pdyb[gsl] = dvy
            pdzb[gsl] = dvz
            rad16 = dvx * dvx + dvy * dvy + dvz * dvz
            for j in range(L):
                e = g * L + j
                radj = rad16[j]
                for v in range(L):
                    sl = pl.ds(v * L, L)
                    xv = (bufa[e, sl] + bufb[e, sl] + bufe[e, sl]
                          + radj * wrv_vals[v])
                    xbuf[e, sl] = xv
                    ss[v] = ss[v] + xv
                    qq[v] = qq[v] + xv * xv
        pltpu.sync_copy(xbuf, x1_o.at[pl.ds(r0, CH1)])
        pltpu.sync_copy(pdxb, pdx_o.at[pl.ds(r0, CH1)])
        pltpu.sync_copy(pdyb, pdy_o.at[pl.ds(r0, CH1)])
        pltpu.sync_copy(pdzb, pdz_o.at[pl.ds(r0, CH1)])
        return (tuple(ss), tuple(qq))

    nch = BASE_CH1 + jnp.where(wid < REM_CH1, 1, 0)
    ss, qq = lax.fori_loop(0, nch, chunk, init)
    for v in range(L):
        ssum[0, v] = ss[v]
        qsum[0, v] = qq[v]
    pltpu.sync_copy(ssum, s1_o.at[pl.ds(wid, 1)])
    pltpu.sync_copy(qsum, q1_o.at[pl.ds(wid, 1)])


def _k1(hwa, hwb, eterm, dsti, srci, w1r, posx, posy, posz):
    mesh = plsc.VectorSubcoreMesh(core_axis_name="c", subcore_axis_name="s")
    f = functools.partial(
        pl.kernel,
        mesh=mesh,
        compiler_params=pltpu.CompilerParams(needs_layout_passes=False),
        out_type=[
            jax.ShapeDtypeStruct((N_EDGES, D), jnp.float32),
            jax.ShapeDtypeStruct((N_EDGES,), jnp.float32),
            jax.ShapeDtypeStruct((N_EDGES,), jnp.float32),
            jax.ShapeDtypeStruct((N_EDGES,), jnp.float32),
            jax.ShapeDtypeStruct((NW, L, L), jnp.float32),
            jax.ShapeDtypeStruct((NW, L, L), jnp.float32),
        ],
        scratch_types=[
            pltpu.VMEM((CH1,), jnp.int32),
            pltpu.VMEM((CH1,), jnp.int32),
            pltpu.VMEM((CH1, D), jnp.float32),
            pltpu.VMEM((CH1, D), jnp.float32),
            pltpu.VMEM((CH1, D), jnp.float32),
            pltpu.VMEM((CH1, D), jnp.float32),
            pltpu.VMEM((CH1,), jnp.float32),
            pltpu.VMEM((CH1,), jnp.float32),
            pltpu.VMEM((CH1,), jnp.float32),
            pltpu.VMEM((N_NODES,), jnp.float32),
            pltpu.VMEM((N_NODES,), jnp.float32),
            pltpu.VMEM((N_NODES,), jnp.float32),
            pltpu.VMEM((L, L), jnp.float32),
            pltpu.VMEM((1, L, L), jnp.float32),
            pltpu.VMEM((1, L, L), jnp.float32),
            pltpu.SemaphoreType.DMA,
            pltpu.SemaphoreType.DMA,
            pltpu.SemaphoreType.DMA,
        ],
    )
    return f(_k1_body)(hwa, hwb, eterm, dsti, srci, w1r, posx, posy, posz)


def _affine_from_stats(s, q, g, be, n):
    mu = jnp.sum(s, axis=0, keepdims=True) / n
    ex2 = jnp.sum(q, axis=0, keepdims=True) / n
    var = ex2 - mu * mu
    a = g * lax.rsqrt(var + EPS)
    c = be - mu * a
    return a, c


def _pad8(x):
    return jnp.concatenate([x, jnp.zeros((7, D), jnp.float32)], axis=0)


def _k2_body(x1_ref, s1_ref, q1_ref, pk_ref, w2_ref, x2_ref, s2_ref, q2_ref):
    i = pl.program_id(0)
    a1, c1 = _affine_from_stats(
        s1_ref[...], q1_ref[...], pk_ref[0:1, :], pk_ref[1:2, :], float(N_EDGES)
    )
    t = jnp.maximum(x1_ref[...] * a1 + c1, 0.0)
    x2 = jnp.dot(t, w2_ref[...], preferred_element_type=jnp.float32) + pk_ref[2:3, :]
    x2_ref[...] = x2
    ps = _pad8(jnp.sum(x2, axis=0, keepdims=True))
    pq = _pad8(jnp.sum(x2 * x2, axis=0, keepdims=True))

    @pl.when(i == 0)
    def _():
        s2_ref[...] = ps
        q2_ref[...] = pq

    @pl.when(i != 0)
    def _():
        s2_ref[...] += ps
        q2_ref[...] += pq


def _k2(x1, s1, q1, pk, w2):
    return pl.pallas_call(
        _k2_body,
        grid=(N_EDGES // TILE_E,),
        in_specs=[
            pl.BlockSpec((TILE_E, D), lambda i: (i, 0)),
            pl.BlockSpec((NW, D), lambda i: (0, 0)),
            pl.BlockSpec((NW, D), lambda i: (0, 0)),
            pl.BlockSpec((8, D), lambda i: (0, 0)),
            pl.BlockSpec((D, D), lambda i: (0, 0)),
        ],
        out_specs=[
            pl.BlockSpec((TILE_E, D), lambda i: (i, 0)),
            pl.BlockSpec((8, D), lambda i: (0, 0)),
            pl.BlockSpec((8, D), lambda i: (0, 0)),
        ],
        out_shape=[
            jax.ShapeDtypeStruct((N_EDGES, D), jnp.float32),
            jax.ShapeDtypeStruct((8, D), jnp.float32),
            jax.ShapeDtypeStruct((8, D), jnp.float32),
        ],
    )(x1, s1, q1, pk, w2)


def _k3_body(x2_ref, s2_ref, q2_ref, pk_ref, wp1_ref, msg_ref, s3_ref, q3_ref):
    i = pl.program_id(0)
    a2, c2 = _affine_from_stats(
        s2_ref[...], q2_ref[...], pk_ref[0:1, :], pk_ref[1:2, :], float(N_EDGES)
    )
    msg = jnp.maximum(x2_ref[...] * a2 + c2, 0.0)
    msg_ref[...] = msg
    y = jnp.dot(msg, wp1_ref[...], preferred_element_type=jnp.float32) + pk_ref[2:3, :]
    ps = _pad8(jnp.sum(y, axis=0, keepdims=True))
    pq = _pad8(jnp.sum(y * y, axis=0, keepdims=True))

    @pl.when(i == 0)
    def _():
        s3_ref[...] = ps
        q3_ref[...] = pq

    @pl.when(i != 0)
    def _():
        s3_ref[...] += ps
        q3_ref[...] += pq


def _k3(x2, s2, q2, pk, wp1):
    return pl.pallas_call(
        _k3_body,
        grid=(N_EDGES // TILE_E,),
        in_specs=[
            pl.BlockSpec((TILE_E, D), lambda i: (i, 0)),
            pl.BlockSpec((8, D), lambda i: (0, 0)),
            pl.BlockSpec((8, D), lambda i: (0, 0)),
            pl.BlockSpec((8, D), lambda i: (0, 0)),
            pl.BlockSpec((D, D), lambda i: (0, 0)),
        ],
        out_specs=[
            pl.BlockSpec((TILE_E, D), lambda i: (i, 0)),
            pl.BlockSpec((8, D), lambda i: (0, 0)),
            pl.BlockSpec((8, D), lambda i: (0, 0)),
        ],
        out_shape=[
            jax.ShapeDtypeStruct((N_EDGES, D), jnp.float32),
            jax.ShapeDtypeStruct((8, D), jnp.float32),
            jax.ShapeDtypeStruct((8, D), jnp.float32),
        ],
    )(x2, s2, q2, pk, wp1)


def _k4_body(msg_ref, s3_ref, q3_ref, pk_ref, wp1_ref, psc_ref):
    a3, c3 = _affine_from_stats(
        s3_ref[...], q3_ref[...], pk_ref[0:1, :], pk_ref[1:2, :], float(N_EDGES)
    )
    y = (
        jnp.dot(msg_ref[...], wp1_ref[...], preferred_element_type=jnp.float32)
        + pk_ref[2:3, :]
    )
    t = jnp.maximum(y * a3 + c3, 0.0)
    psc = jnp.sum(t * pk_ref[3:4, :], axis=1, keepdims=True) + pk_ref[4:5, 0:1]
    psc_ref[...] = jnp.broadcast_to(psc, (psc.shape[0], 8))


def _k4(msg, s3, q3, pk, wp1):
    return pl.pallas_call(
        _k4_body,
        grid=(N_EDGES // TILE_E,),
        in_specs=[
            pl.BlockSpec((TILE_E, D), lambda i: (i, 0)),
            pl.BlockSpec((8, D), lambda i: (0, 0)),
            pl.BlockSpec((8, D), lambda i: (0, 0)),
            pl.BlockSpec((8, D), lambda i: (0, 0)),
            pl.BlockSpec((D, D), lambda i: (0, 0)),
        ],
        out_specs=pl.BlockSpec((TILE_E, 8), lambda i: (i, 0)),
        out_shape=jax.ShapeDtypeStruct((N_EDGES, 8), jnp.float32),
    )(msg, s3, q3, pk, wp1)


NR = 320
ACC_R = NR + 8
TRASH_R = NR
CHD = 1280
NCHD = N_EDGES // CHD
GB = 32
CAP = CHD + GB
NPS = 640
NP = NS * NPS
PB = 4 * NP
PR = NPS + 16
CH5P = 1280
NCH5P = N_EDGES // CH5P


def _k5_body(msg, pdx, pdy, pdz, psc, dsti,
             aggm_o, pb_o,
             acc, gbuf, dbufs, cbuf, lbuf, dbuf2, pdxv, pdyv, pdzv, pscv, pall,
             gsem):
    cid = lax.axis_index("c")
    sid = lax.axis_index("s")
    wid2 = cid * NS + sid
    nbase = wid2 * NR

    zero = jnp.zeros((L,), jnp.float32)
    iota16 = lax.iota(jnp.int32, L)
    trash16 = jnp.full((L,), TRASH_R, jnp.int32)
    zeros16i = jnp.zeros((L,), jnp.int32)

    def zacc(i, _):
        for v in range(D // L):
            acc[i, pl.ds(v * L, L)] = zero
        return 0

    lax.fori_loop(0, ACC_R, zacc, 0)

    def zp(i, _):
        pall[pl.ds(i * L, L)] = zero
        return 0

    lax.fori_loop(0, (4 * PR) // L, zp, 0)

    def chunk(ci, _):
        r0 = ci * CHD
        pltpu.sync_copy(dsti.at[pl.ds(r0, CHD)], dbufs)

        def scan(g, cursor):
            gsl = pl.ds(g * L, L)
            dv = dbufs[gsl]
            local = dv - nbase
            inr = (local >= 0) & (local < NR)
            eids = iota16 + (r0 + g * L)
            plsc.store_compressed(cbuf.at[pl.ds(cursor, L)], eids, mask=inr)
            plsc.store_compressed(lbuf.at[pl.ds(cursor, L)], local, mask=inr)
            cnt = plsc.all_reduce_population_count(inr)[0]
            return cursor + cnt

        m = lax.fori_loop(0, CHD // L, scan, jnp.int32(0))
        lbuf[pl.ds(m, L)] = trash16
        lbuf[pl.ds(m + L, L)] = trash16
        cbuf[pl.ds(m, L)] = zeros16i
        cbuf[pl.ds(m + L, L)] = zeros16i
        nb = (m + GB - 1) // GB

        def batch(b, _):
            cp = pltpu.async_copy(msg.at[cbuf.at[pl.ds(b * GB, GB)]], gbuf,
                                  gsem)
            cp.wait()
            for g2 in range(GB // L):
                lv = lbuf[pl.ds(b * GB + g2 * L, L)]
                for j in range(L):
                    e = g2 * L + j
                    rowv = jnp.full((L,), lv[j], jnp.int32)
                    for seg in range(D // L):
                        plsc.addupdate_scatter(
                            acc, [rowv, iota16 + seg * L],
                            gbuf[e, pl.ds(seg * L, L)])
            return 0

        lax.fori_loop(0, nb, batch, 0)
        return 0

    lax.fori_loop(0, NCHD, chunk, 0)

    @pl.when(wid2 < NC * NS - 1)
    def _():
        pltpu.sync_copy(acc.at[pl.ds(0, NR)], aggm_o.at[pl.ds(nbase, NR)])

    @pl.when(wid2 == NC * NS - 1)
    def _():
        nlast = N_NODES - (NC * NS - 1) * NR
        pltpu.sync_copy(acc.at[pl.ds(0, nlast)],
                        aggm_o.at[pl.ds(nbase, nlast)])

    @pl.when(cid == 0)
    def _():
        pbase = sid * NPS
        ones16 = zero + 1.0

        def pchunk(ci, _):
            r0 = ci * CH5P
            pltpu.sync_copy(dsti.at[pl.ds(r0, CH5P)], dbuf2)
            pltpu.sync_copy(pdx.at[pl.ds(r0, CH5P)], pdxv)
            pltpu.sync_copy(pdy.at[pl.ds(r0, CH5P)], pdyv)
            pltpu.sync_copy(pdz.at[pl.ds(r0, CH5P)], pdzv)
            pltpu.sync_copy(psc.at[pl.ds(r0, CH5P)], pscv)

            def pgroup(g, _):
                gsl = pl.ds(g * L, L)
                dv = dbuf2[gsl]
                local = dv - pbase
                inr = (local >= 0) & (local < NPS)
                idx = jnp.where(inr, local, NPS)
                ps = pscv[gsl]
                plsc.addupdate_scatter(pall, [idx], pdxv[gsl] * ps)
                plsc.addupdate_scatter(pall, [idx + PR], pdyv[gsl] * ps)
                plsc.addupdate_scatter(pall, [idx + 2 * PR], pdzv[gsl] * ps)
                plsc.addupdate_scatter(pall, [idx + 3 * PR], ones16)
                return 0

            lax.fori_loop(0, CH5P // L, pgroup, 0)
            return 0

        lax.fori_loop(0, NCH5P, pchunk, 0)
        for comp in range(4):
            pltpu.sync_copy(pall.at[pl.ds(comp * PR, NPS)],
                            pb_o.at[pl.ds(comp * NP + sid * NPS, NPS)])


def _k5(msg, pdx, pdy, pdz, psc, dsti):
    mesh = plsc.VectorSubcoreMesh(core_axis_name="c", subcore_axis_name="s")
    f = functools.partial(
        pl.kernel,
        mesh=mesh,
        compiler_params=pltpu.CompilerParams(needs_layout_passes=False),
        out_type=[
            jax.ShapeDtypeStruct((N_NODES, D), jnp.float32),
            jax.ShapeDtypeStruct((PB,), jnp.float32),
        ],
        scratch_types=[
            pltpu.VMEM((ACC_R, D), jnp.float32),
            pltpu.VMEM((GB, D), jnp.float32),
            pltpu.VMEM((CHD,), jnp.int32),
            pltpu.VMEM((CAP,), jnp.int32),
            pltpu.VMEM((CAP,), jnp.int32),
            pltpu.VMEM((CH5P,), jnp.int32),
            pltpu.VMEM((CH5P,), jnp.float32),
            pltpu.VMEM((CH5P,), jnp.float32),
            pltpu.VMEM((CH5P,), jnp.float32),
            pltpu.VMEM((CH5P,), jnp.float32),
            pltpu.VMEM((4 * PR,), jnp.float32),
            pltpu.SemaphoreType.DMA,
        ],
    )
    return f(_k5_body)(msg, pdx, pdy, pdz, psc, dsti)


def _k6a_body(h_ref, m_ref, wa_ref, wb_ref, pk_ref, u1_ref, s_ref, q_ref):
    i = pl.program_id(0)
    u1 = (
        jnp.dot(h_ref[...], wa_ref[...], preferred_element_type=jnp.float32)
        + jnp.dot(m_ref[...], wb_ref[...], preferred_element_type=jnp.float32)
        + pk_ref[0:1, :]
    )
    u1_ref[...] = u1
    ps = _pad8(jnp.sum(u1, axis=0, keepdims=True))
    pq = _pad8(jnp.sum(u1 * u1, axis=0, keepdims=True))

    @pl.when(i == 0)
    def _():
        s_ref[...] = ps
        q_ref[...] = pq

    @pl.when(i != 0)
    def _():
        s_ref[...] += ps
        q_ref[...] += pq


def _k6a(h, aggm, wa, wb, pk):
    return pl.pallas_call(
        _k6a_body,
        grid=(N_NODES // TILE_N,),
        in_specs=[
            pl.BlockSpec((TILE_N, D), lambda i: (i, 0)),
            pl.BlockSpec((TILE_N, D), lambda i: (i, 0)),
            pl.BlockSpec((D, D), lambda i: (0, 0)),
            pl.BlockSpec((D, D), lambda i: (0, 0)),
            pl.BlockSpec((8, D), lambda i: (0, 0)),
        ],
        out_specs=[
            pl.BlockSpec((TILE_N, D), lambda i: (i, 0)),
            pl.BlockSpec((8, D), lambda i: (0, 0)),
            pl.BlockSpec((8, D), lambda i: (0, 0)),
        ],
        out_shape=[
            jax.ShapeDtypeStruct((N_NODES, D), jnp.float32),
            jax.ShapeDtypeStruct((8, D), jnp.float32),
            jax.ShapeDtypeStruct((8, D), jnp.float32),
        ],
    )(h, aggm, wa, wb, pk)


def _k6b_body(u1_ref, s_ref, q_ref, pk_ref, w_ref, u2_ref, s2_ref, q2_ref):
    i = pl.program_id(0)
    a, c = _affine_from_stats(
        s_ref[...], q_ref[...], pk_ref[0:1, :], pk_ref[1:2, :], float(N_NODES)
    )
    t = jnp.maximum(u1_ref[...] * a + c, 0.0)
    u2 = jnp.dot(t, w_ref[...], preferred_element_type=jnp.float32) + pk_ref[2:3, :]
    u2_ref[...] = u2
    ps = _pad8(jnp.sum(u2, axis=0, keepdims=True))
    pq = _pad8(jnp.sum(u2 * u2, axis=0, keepdims=True))

    @pl.when(i == 0)
    def _():
        s2_ref[...] = ps
        q2_ref[...] = pq

    @pl.when(i != 0)
    def _():
        s2_ref[...] += ps
        q2_ref[...] += pq


def _k6b(u1, s, q, pk, w):
    return pl.pallas_call(
        _k6b_body,
        grid=(N_NODES // TILE_N,),
        in_specs=[
            pl.BlockSpec((TILE_N, D), lambda i: (i, 0)),
            pl.BlockSpec((8, D), lambda i: (0, 0)),
            pl.BlockSpec((8, D), lambda i: (0, 0)),
            pl.BlockSpec((8, D), lambda i: (0, 0)),
            pl.BlockSpec((D, D), lambda i: (0, 0)),
        ],
        out_specs=[
            pl.BlockSpec((TILE_N, D), lambda i: (i, 0)),
            pl.BlockSpec((8, D), lambda i: (0, 0)),
            pl.BlockSpec((8, D), lambda i: (0, 0)),
        ],
        out_shape=[
            jax.ShapeDtypeStruct((N_NODES, D), jnp.float32),
            jax.ShapeDtypeStruct((8, D), jnp.float32),
            jax.ShapeDtypeStruct((8, D), jnp.float32),
        ],
    )(u1, s, q, pk, w)


def _k6c_body(u2_ref, s_ref, q_ref, pk_ref, pos_ref, psum_ref, cnt_ref,
              ho_ref, po_ref):
    a, c = _affine_from_stats(
        s_ref[...], q_ref[...], pk_ref[0:1, :], pk_ref[1:2, :], float(N_NODES)
    )
    ho_ref[...] = jnp.maximum(u2_ref[...] * a + c, 0.0)
    cnt = jnp.maximum(cnt_ref[...], 1.0)
    po_ref[...] = pos_ref[...] + psum_ref[...] / cnt


def _k6c(u2, s, q, pk, pos, psum, cnt):
    return pl.pallas_call(
        _k6c_body,
        grid=(N_NODES // TILE_N,),
        in_specs=[
            pl.BlockSpec((TILE_N, D), lambda i: (i, 0)),
            pl.BlockSpec((8, D), lambda i: (0, 0)),
            pl.BlockSpec((8, D), lambda i: (0, 0)),
            pl.BlockSpec((8, D), lambda i: (0, 0)),
            pl.BlockSpec((TILE_N, 3), lambda i: (i, 0)),
            pl.BlockSpec((TILE_N, 3), lambda i: (i, 0)),
            pl.BlockSpec((TILE_N, 1), lambda i: (i, 0)),
        ],
        out_specs=[
            pl.BlockSpec((TILE_N, D), lambda i: (i, 0)),
            pl.BlockSpec((TILE_N, 3), lambda i: (i, 0)),
        ],
        out_shape=[
            jax.ShapeDtypeStruct((N_NODES, D), jnp.float32),
            jax.ShapeDtypeStruct((N_NODES, 3), jnp.float32),
        ],
    )(u2, s, q, pk, pos, psum, cnt)


def kernel(h, pos, edge_index, edge_attr, params):
    p = params
    src = edge_index[0].astype(jnp.int32)
    dst = edge_index[1].astype(jnp.int32)

    w1 = p["W_msg1"]
    wa, wb = w1[0:D], w1[D:2 * D]
    w1r = w1[2 * D].reshape(L, L)
    we = w1[2 * D + 1:]

    z = jnp.zeros((D,), jnp.float32)

    def pack(*rows):
        return jnp.stack(list(rows) + [z] * (8 - len(rows)))

    posx = pos[:, 0] + 0.0
    posy = pos[:, 1] + 0.0
    posz = pos[:, 2] + 0.0

    hwa, hwb = _k0a(h, wa, wb)
    eterm = _k0b(edge_attr, we, pack(p["b_msg1"]))
    x1, pdx, pdy, pdz, s1, q1 = _k1(hwa, hwb, eterm, dst, src, w1r,
                                    posx, posy, posz)
    s1 = s1.reshape(NW, D)
    q1 = q1.reshape(NW, D)
    x2, s2, q2 = _k2(x1, s1, q1, pack(p["g_msg1"], p["be_msg1"], p["b_msg2"]),
                     p["W_msg2"])
    msg, s3, q3 = _k3(x2, s2, q2, pack(p["g_msg2"], p["be_msg2"], p["b_pos1"]),
                      p["W_pos1"])
    psc8 = _k4(msg, s3, q3,
               pack(p["g_pos1"], p["be_pos1"], p["b_pos1"], p["W_pos2"][:, 0],
                    jnp.full((D,), p["b_pos2"][0])),
               p["W_pos1"])
    psc = psc8[:, 0]
    aggm, pb = _k5(msg, pdx, pdy, pdz, psc, dst)
    pb2 = pb.reshape(4, NP)
    psum = jnp.stack([pb2[0, :N_NODES], pb2[1, :N_NODES], pb2[2, :N_NODES]],
                     axis=1)
    cnt = pb2[3, :N_NODES].reshape(N_NODES, 1)

    wu = p["W_upd1"]
    u1, s4, q4 = _k6a(h, aggm, wu[0:D], wu[D:2 * D], pack(p["b_upd1"]))
    u2, s5, q5 = _k6b(u1, s4, q4, pack(p["g_upd1"], p["be_upd1"], p["b_upd2"]),
                      p["W_upd2"])
    h_out, pos_out = _k6c(u2, s5, q5, pack(p["g_upd2"], p["be_upd2"]), pos,
                          psum, cnt)
    return (h_out, pos_out)
